# baseline (speedup 1.0000x reference)
import jax
import jax.numpy as jnp
from jax.experimental import pallas as pl
from jax.experimental.pallas import tpu as pltpu

_N = 10000
_E = 160000
_D = 256
_ROWS = 1000
_NBLK = _N // _ROWS
_ECH = 2000
_NECH = _E // _ECH


def _linear(x, w, b, relu):
    def body(x_ref, w_ref, b_ref, o_ref):
        y = jnp.dot(x_ref[...], w_ref[...],
                    preferred_element_type=jnp.float32) + b_ref[...]
        if relu:
            y = jnp.maximum(y, 0.0)
        o_ref[...] = y

    return pl.pallas_call(
        body,
        grid=(_NBLK,),
        in_specs=[
            pl.BlockSpec((_ROWS, _D), lambda i: (i, 0)),
            pl.BlockSpec((_D, _D), lambda i: (0, 0)),
            pl.BlockSpec((1, _D), lambda i: (0, 0)),
        ],
        out_specs=pl.BlockSpec((_ROWS, _D), lambda i: (i, 0)),
        out_shape=jax.ShapeDtypeStruct((_N, _D), jnp.float32),
    )(x, w, b.reshape(1, _D))


def _segmax(t, src, dst):
    src3 = src.reshape(_NECH, 1, _ECH)
    dst3 = dst.reshape(_NECH, 1, _ECH)

    def body(src_ref, dst_ref, t_ref, agg_ref):
        @pl.when(pl.program_id(0) == 0)
        def _():
            agg_ref[...] = jnp.zeros_like(agg_ref)

        def edge(i, carry):
            s = src_ref[0, 0, i]
            d = dst_ref[0, 0, i]
            agg_ref[d, :] = jnp.maximum(agg_ref[d, :], t_ref[s, :])
            return carry

        jax.lax.fori_loop(0, _ECH, edge, 0)

    return pl.pallas_call(
        body,
        grid=(_NECH,),
        in_specs=[
            pl.BlockSpec((1, 1, _ECH), lambda i: (i, 0, 0),
                         memory_space=pltpu.SMEM),
            pl.BlockSpec((1, 1, _ECH), lambda i: (i, 0, 0),
                         memory_space=pltpu.SMEM),
            pl.BlockSpec((_N, _D), lambda i: (0, 0)),
        ],
        out_specs=pl.BlockSpec((_N, _D), lambda i: (0, 0)),
        out_shape=jax.ShapeDtypeStruct((_N, _D), jnp.float32),
    )(src3, dst3, t)


def _layer0_update(f0, agg0, w_bot):
    def body(f0_ref, agg_ref, w_ref, u_ref, ps_ref, pss_ref):
        y = f0_ref[...] + jnp.dot(agg_ref[...], w_ref[...],
                                  preferred_element_type=jnp.float32)
        u = jnp.maximum(y, 0.0)
        u_ref[...] = u
        ps_ref[0, ...] = jnp.sum(u, axis=0, keepdims=True)
        pss_ref[0, ...] = jnp.sum(u * u, axis=0, keepdims=True)

    return pl.pallas_call(
        body,
        grid=(_NBLK,),
        in_specs=[
            pl.BlockSpec((_ROWS, _D), lambda i: (i, 0)),
            pl.BlockSpec((_ROWS, _D), lambda i: (i, 0)),
            pl.BlockSpec((_D, _D), lambda i: (0, 0)),
        ],
        out_specs=[
            pl.BlockSpec((_ROWS, _D), lambda i: (i, 0)),
            pl.BlockSpec((1, 1, _D), lambda i: (i, 0, 0)),
            pl.BlockSpec((1, 1, _D), lambda i: (i, 0, 0)),
        ],
        out_shape=[
            jax.ShapeDtypeStruct((_N, _D), jnp.float32),
            jax.ShapeDtypeStruct((_NBLK, 1, _D), jnp.float32),
            jax.ShapeDtypeStruct((_NBLK, 1, _D), jnp.float32),
        ],
    )(f0, agg0, w_bot)


def _bn_norm_t1(u, ps, pss, gamma, beta, Wp1, bp1):
    def body(u_ref, ps_ref, pss_ref, g_ref, be_ref, w_ref, b_ref,
             h_ref, t1_ref):
        mean = jnp.sum(ps_ref[...], axis=0) / _N
        var = jnp.sum(pss_ref[...], axis=0) / _N - mean * mean
        inv = jax.lax.rsqrt(var + 1e-5)
        hb = (u_ref[...] - mean) * inv * g_ref[...] + be_ref[...]
        norm = jnp.sqrt(jnp.sum(hb * hb, axis=1, keepdims=True))
        hn = hb / (norm + 1e-6)
        h_ref[...] = hn
        t1_ref[...] = jnp.maximum(
            jnp.dot(hn, w_ref[...], preferred_element_type=jnp.float32)
            + b_ref[...], 0.0)

    return pl.pallas_call(
        body,
        grid=(_NBLK,),
        in_specs=[
            pl.BlockSpec((_ROWS, _D), lambda i: (i, 0)),
            pl.BlockSpec((_NBLK, 1, _D), lambda i: (0, 0, 0)),
            pl.BlockSpec((_NBLK, 1, _D), lambda i: (0, 0, 0)),
            pl.BlockSpec((1, _D), lambda i: (0, 0)),
            pl.BlockSpec((1, _D), lambda i: (0, 0)),
            pl.BlockSpec((_D, _D), lambda i: (0, 0)),
            pl.BlockSpec((1, _D), lambda i: (0, 0)),
        ],
        out_specs=[
            pl.BlockSpec((_ROWS, _D), lambda i: (i, 0)),
            pl.BlockSpec((_ROWS, _D), lambda i: (i, 0)),
        ],
        out_shape=[
            jax.ShapeDtypeStruct((_N, _D), jnp.float32),
            jax.ShapeDtypeStruct((_N, _D), jnp.float32),
        ],
    )(u, ps, pss, gamma.reshape(1, _D), beta.reshape(1, _D), Wp1,
      bp1.reshape(1, _D))


def _final(h1, agg1, w_bot):
    def body(h1_ref, agg_ref, w_ref, o_ref):
        o_ref[...] = h1_ref[...] + jnp.dot(
            agg_ref[...], w_ref[...], preferred_element_type=jnp.float32)

    return pl.pallas_call(
        body,
        grid=(_NBLK,),
        in_specs=[
            pl.BlockSpec((_ROWS, _D), lambda i: (i, 0)),
            pl.BlockSpec((_ROWS, _D), lambda i: (i, 0)),
            pl.BlockSpec((_D, _D), lambda i: (0, 0)),
        ],
        out_specs=pl.BlockSpec((_ROWS, _D), lambda i: (i, 0)),
        out_shape=jax.ShapeDtypeStruct((_N, _D), jnp.float32),
    )(h1, agg1, w_bot)


def kernel(features, edge_index, Wp0, bp0, Wp1, bp1, Wfc0, bfc0, Wfc1, bfc1,
           gamma0, beta0):
    src = edge_index[0]
    dst = edge_index[1]
    t0 = _linear(features, Wp0, bp0, relu=True)
    f0 = _linear(features, Wfc0[:_D], bfc0, relu=False)
    agg0 = _segmax(t0, src, dst)
    u, ps, pss = _layer0_update(f0, agg0, Wfc0[_D:])
    h, t1 = _bn_norm_t1(u, ps, pss, gamma0, beta0, Wp1, bp1)
    h1 = _linear(h, Wfc1[:_D], bfc1, relu=False)
    agg1 = _segmax(t1, src, dst)
    out = _final(h1, agg1, Wfc1[_D:])
    return out


# speedup vs baseline: 1.6701x; 1.6701x over previous
import dataclasses
import functools

import jax
import jax.numpy as jnp
from jax import lax
from jax.experimental import pallas as pl
from jax.experimental.pallas import tpu as pltpu
from jax.experimental.pallas import tpu_sc as plsc

_N = 10000
_E = 160000
_D = 256
_ROWS = 1000
_NBLK = _N // _ROWS
_ECH = 2000
_NECH = _E // _ECH

_NW = 32
_RPT = 320
_NPAD = _NW * _RPT
_CAP = 128
_SECH = 2000
_GRP = _SECH // 16


def _linear(x, w, b, relu):
    def body(x_ref, w_ref, b_ref, o_ref):
        y = jnp.dot(x_ref[...], w_ref[...],
                    preferred_element_type=jnp.float32) + b_ref[...]
        if relu:
            y = jnp.maximum(y, 0.0)
        o_ref[...] = y

    return pl.pallas_call(
        body,
        grid=(_NBLK,),
        in_specs=[
            pl.BlockSpec((_ROWS, _D), lambda i: (i, 0)),
            pl.BlockSpec((_D, _D), lambda i: (0, 0)),
            pl.BlockSpec((1, _D), lambda i: (0, 0)),
        ],
        out_specs=pl.BlockSpec((_ROWS, _D), lambda i: (i, 0)),
        out_shape=jax.ShapeDtypeStruct((_N, _D), jnp.float32),
    )(x, w, b.reshape(1, _D))


def _segmax_body(t_hbm, src_hbm, dst_hbm, out_hbm,
                 aggbuf, rowbuf, bsrc, bdst, esrc, edst, sem):
    wid = lax.axis_index("c") * 16 + lax.axis_index("s")
    lo = wid * _RPT

    zf = jnp.zeros((16,), jnp.float32)
    lane = lax.iota(jnp.int32, 16)

    @pl.loop(0, _RPT + 1)
    def _(r):
        for c in range(0, _D, 16):
            aggbuf[r, pl.ds(c, 16)] = zf

    @pl.loop(0, _CAP, step=16)
    def _(i):
        bsrc[pl.ds(i, 16)] = jnp.zeros((16,), jnp.int32)
        bdst[pl.ds(i, 16)] = jnp.full((16,), _RPT, jnp.int32)

    def flush():
        pltpu.async_copy(t_hbm.at[bsrc], rowbuf, sem).wait()

        @pl.loop(0, _CAP // 16)
        def _(g):
            dgrp = bdst[pl.ds(g * 16, 16)]

            @pl.loop(0, 16)
            def _(l):
                d = jnp.max(jnp.where(lane == l, dgrp, 0))
                i = g * 16 + l
                for c in range(0, _D, 16):
                    v = jnp.maximum(aggbuf[d, pl.ds(c, 16)],
                                    rowbuf[i, pl.ds(c, 16)])
                    aggbuf[d, pl.ds(c, 16)] = v

    def group(g, cnt):
        svec = esrc[pl.ds(g * 16, 16)]
        dvec = edst[pl.ds(g * 16, 16)]
        mask = (dvec >= lo) & (dvec < lo + _RPT)
        npop = jnp.sum(jnp.where(mask, 1, 0))

        def do_flush():
            flush()
            return 0

        cnt = lax.cond(cnt > _CAP - 16, do_flush, lambda: cnt)
        plsc.store_compressed(bsrc.at[pl.ds(cnt, 16)], svec, mask=mask)
        plsc.store_compressed(bdst.at[pl.ds(cnt, 16)], dvec - lo, mask=mask)
        return cnt + npop

    def chunk(ch, cnt):
        pltpu.sync_copy(src_hbm.at[pl.ds(ch * _SECH, _SECH)], esrc)
        pltpu.sync_copy(dst_hbm.at[pl.ds(ch * _SECH, _SECH)], edst)
        return lax.fori_loop(0, _GRP, group, cnt)

    lax.fori_loop(0, _E // _SECH, chunk, 0)
    flush()

    pltpu.sync_copy(aggbuf.at[pl.ds(0, _RPT)], out_hbm.at[pl.ds(lo, _RPT)])


def _segmax(t, src, dst):
    cp = pltpu.CompilerParams()
    if "needs_layout_passes" in pltpu.CompilerParams.__dataclass_fields__:
        cp = dataclasses.replace(cp, needs_layout_passes=False)
    k = pl.kernel(
        _segmax_body,
        out_type=jax.ShapeDtypeStruct((_NPAD, _D), jnp.float32),
        compiler_params=cp,
        mesh=plsc.VectorSubcoreMesh(core_axis_name="c", subcore_axis_name="s"),
        scratch_types=[
            pltpu.VMEM((_RPT + 1, _D), jnp.float32),
            pltpu.VMEM((_CAP, _D), jnp.float32),
            pltpu.VMEM((_CAP,), jnp.int32),
            pltpu.VMEM((_CAP,), jnp.int32),
            pltpu.VMEM((_SECH,), jnp.int32),
            pltpu.VMEM((_SECH,), jnp.int32),
            pltpu.SemaphoreType.DMA,
        ],
    )
    return k(t, src, dst)[:_N]


def _layer0_update(f0, agg0, w_bot):
    def body(f0_ref, agg_ref, w_ref, u_ref, ps_ref, pss_ref):
        y = f0_ref[...] + jnp.dot(agg_ref[...], w_ref[...],
                                  preferred_element_type=jnp.float32)
        u = jnp.maximum(y, 0.0)
        u_ref[...] = u
        ps_ref[0, ...] = jnp.sum(u, axis=0, keepdims=True)
        pss_ref[0, ...] = jnp.sum(u * u, axis=0, keepdims=True)

    return pl.pallas_call(
        body,
        grid=(_NBLK,),
        in_specs=[
            pl.BlockSpec((_ROWS, _D), lambda i: (i, 0)),
            pl.BlockSpec((_ROWS, _D), lambda i: (i, 0)),
            pl.BlockSpec((_D, _D), lambda i: (0, 0)),
        ],
        out_specs=[
            pl.BlockSpec((_ROWS, _D), lambda i: (i, 0)),
            pl.BlockSpec((1, 1, _D), lambda i: (i, 0, 0)),
            pl.BlockSpec((1, 1, _D), lambda i: (i, 0, 0)),
        ],
        out_shape=[
            jax.ShapeDtypeStruct((_N, _D), jnp.float32),
            jax.ShapeDtypeStruct((_NBLK, 1, _D), jnp.float32),
            jax.ShapeDtypeStruct((_NBLK, 1, _D), jnp.float32),
        ],
    )(f0, agg0, w_bot)


def _bn_norm_t1(u, ps, pss, gamma, beta, Wp1, bp1):
    def body(u_ref, ps_ref, pss_ref, g_ref, be_ref, w_ref, b_ref,
             h_ref, t1_ref):
        mean = jnp.sum(ps_ref[...], axis=0) / _N
        var = jnp.sum(pss_ref[...], axis=0) / _N - mean * mean
        inv = jax.lax.rsqrt(var + 1e-5)
        hb = (u_ref[...] - mean) * inv * g_ref[...] + be_ref[...]
        norm = jnp.sqrt(jnp.sum(hb * hb, axis=1, keepdims=True))
        hn = hb / (norm + 1e-6)
        h_ref[...] = hn
        t1_ref[...] = jnp.maximum(
            jnp.dot(hn, w_ref[...], preferred_element_type=jnp.float32)
            + b_ref[...], 0.0)

    return pl.pallas_call(
        body,
        grid=(_NBLK,),
        in_specs=[
            pl.BlockSpec((_ROWS, _D), lambda i: (i, 0)),
            pl.BlockSpec((_NBLK, 1, _D), lambda i: (0, 0, 0)),
            pl.BlockSpec((_NBLK, 1, _D), lambda i: (0, 0, 0)),
            pl.BlockSpec((1, _D), lambda i: (0, 0)),
            pl.BlockSpec((1, _D), lambda i: (0, 0)),
            pl.BlockSpec((_D, _D), lambda i: (0, 0)),
            pl.BlockSpec((1, _D), lambda i: (0, 0)),
        ],
        out_specs=[
            pl.BlockSpec((_ROWS, _D), lambda i: (i, 0)),
            pl.BlockSpec((_ROWS, _D), lambda i: (i, 0)),
        ],
        out_shape=[
            jax.ShapeDtypeStruct((_N, _D), jnp.float32),
            jax.ShapeDtypeStruct((_N, _D), jnp.float32),
        ],
    )(u, ps, pss, gamma.reshape(1, _D), beta.reshape(1, _D), Wp1,
      bp1.reshape(1, _D))


def _final(h1, agg1, w_bot):
    def body(h1_ref, agg_ref, w_ref, o_ref):
        o_ref[...] = h1_ref[...] + jnp.dot(
            agg_ref[...], w_ref[...], preferred_element_type=jnp.float32)

    return pl.pallas_call(
        body,
        grid=(_NBLK,),
        in_specs=[
            pl.BlockSpec((_ROWS, _D), lambda i: (i, 0)),
            pl.BlockSpec((_ROWS, _D), lambda i: (i, 0)),
            pl.BlockSpec((_D, _D), lambda i: (0, 0)),
        ],
        out_specs=pl.BlockSpec((_ROWS, _D), lambda i: (i, 0)),
        out_shape=jax.ShapeDtypeStruct((_N, _D), jnp.float32),
    )(h1, agg1, w_bot)


def kernel(features, edge_index, Wp0, bp0, Wp1, bp1, Wfc0, bfc0, Wfc1, bfc1,
           gamma0, beta0):
    src = edge_index[0]
    dst = edge_index[1]
    t0 = _linear(features, Wp0, bp0, relu=True)
    f0 = _linear(features, Wfc0[:_D], bfc0, relu=False)
    agg0 = _segmax(t0, src, dst)
    u, ps, pss = _layer0_update(f0, agg0, Wfc0[_D:])
    h, t1 = _bn_norm_t1(u, ps, pss, gamma0, beta0, Wp1, bp1)
    h1 = _linear(h, Wfc1[:_D], bfc1, relu=False)
    agg1 = _segmax(t1, src, dst)
    out = _final(h1, agg1, Wfc1[_D:])
    return out
